# baseline (speedup 1.0000x reference)
import functools

import jax
import jax.numpy as jnp
from jax import lax
from jax.experimental import pallas as pl
from jax.experimental.pallas import tpu as pltpu
from jax.experimental.pallas import tpu_sc as plsc

_NC = 2
_NS = 16
_NW = _NC * _NS
_LANES = 16


def _sc_body(table_hbm, idx_hbm, out_hbm, spmem_table, idx_v, sem_out,
             *, n, vocab):
    sid = lax.axis_index("s")
    wid = sid * _NC + lax.axis_index("c")
    pos_per_w = n // _NW
    base = wid * pos_per_w

    rows_per_tile = vocab // _NS
    pltpu.sync_copy(table_hbm.at[pl.ds(sid * rows_per_tile, rows_per_tile)],
                    spmem_table.at[pl.ds(sid * rows_per_tile, rows_per_tile)])
    pltpu.sync_copy(idx_hbm.at[pl.ds(base, pos_per_w)], idx_v)
    plsc.subcore_barrier()

    def drain_one():
        pltpu.make_async_copy(
            spmem_table.at[0], out_hbm.at[0], sem_out).wait()

    nvec = pos_per_w // _LANES

    def push_vec(v, _):
        off = pl.multiple_of(v * _LANES, _LANES)
        lvec = idx_v[pl.ds(off, _LANES)]
        for lane in range(_LANES):
            pltpu.async_copy(spmem_table.at[lvec[lane]],
                             out_hbm.at[base + off + lane], sem_out)

        @pl.when(v >= 1)
        def _lag():
            for _ in range(_LANES):
                drain_one()

        return _

    pl.loop(0, nvec, init_carry=jnp.int32(0))(push_vec)
    for _ in range(_LANES):
        drain_one()


def kernel(prefix, table):
    bsz, seq = prefix.shape
    n = bsz * seq
    vocab, width = table.shape

    idx = prefix.reshape(n).astype(jnp.int32)
    mesh = plsc.VectorSubcoreMesh(core_axis_name="c", subcore_axis_name="s")
    body = functools.partial(_sc_body, n=n, vocab=vocab)
    k = pl.kernel(
        body,
        out_type=jax.ShapeDtypeStruct((n, width), table.dtype),
        mesh=mesh,
        compiler_params=pltpu.CompilerParams(needs_layout_passes=False),
        scratch_types=[
            pltpu.VMEM_SHARED((vocab, width), table.dtype),
            pltpu.VMEM((n // _NW,), jnp.int32),
            pltpu.SemaphoreType.DMA,
        ],
    )
    out = k(table, idx)
    return out.reshape(bsz, seq, width)
